# baseline (speedup 1.0000x reference)
import functools

import jax
import jax.numpy as jnp
from jax import lax
from jax.experimental import pallas as pl
from jax.experimental.pallas import tpu as pltpu
from jax.experimental.pallas import tpu_sc as plsc

N = 10000
D = 128
DA = 136
NC = 2
NS = 16
NW = NC * NS
C = 128
GS = 8
NP = 10112
SP = NP // NS


def _sc_segment_sum(table, src_w, dst_w, init):
    kc = src_w.shape[1]
    ng = kc // GS
    mesh = plsc.VectorSubcoreMesh(core_axis_name="c", subcore_axis_name="s")

    @functools.partial(
        pl.kernel,
        out_type=jax.ShapeDtypeStruct((NC, NP, DA), jnp.float32),
        mesh=mesh,
        compiler_params=pltpu.CompilerParams(use_tc_tiling_on_sc=False),
        scratch_types=[
            pltpu.VMEM((2, GS, C), jnp.int32),
            pltpu.VMEM((2, GS, C), jnp.int32),
            pltpu.VMEM((2, C, DA), jnp.float32),
            pltpu.VMEM_SHARED((NP, DA), jnp.float32),
            pltpu.SemaphoreType.DMA,
            pltpu.SemaphoreType.DMA,
        ],
    )
    def seg_sum(table_hbm, src_hbm, dst_hbm, init_hbm, out_hbm,
                src_v, dst_v, rows_v, acc, sem, sem_i):
        cid = lax.axis_index("c")
        sid = lax.axis_index("s")
        wid = cid * NS + sid

        pltpu.sync_copy(init_hbm.at[cid, pl.ds(sid * SP, SP)],
                        acc.at[pl.ds(sid * SP, SP)])
        pltpu.sync_copy(src_hbm.at[wid, pl.ds(0, GS)], src_v.at[0])
        pltpu.sync_copy(dst_hbm.at[wid, pl.ds(0, GS)], dst_v.at[0])
        plsc.subcore_barrier()

        def g_start(ib, j, buf):
            pltpu.async_copy(table_hbm.at[src_v.at[ib, j]], rows_v.at[buf],
                             sem)

        def g_wait(ib, j, buf):
            pltpu.make_async_copy(
                table_hbm.at[src_v.at[ib, j]], rows_v.at[buf], sem).wait()

        def scat(ib, j, buf):
            pltpu.sync_copy(rows_v.at[buf], acc.at[dst_v.at[ib, j]], add=True)

        def group(g, _):
            ib = lax.rem(g, 2)
            nib = 1 - ib

            @pl.when(g + 1 < ng)
            def _prefetch():
                pltpu.async_copy(src_hbm.at[wid, pl.ds((g + 1) * GS, GS)],
                                 src_v.at[nib], sem_i)
                pltpu.async_copy(dst_hbm.at[wid, pl.ds((g + 1) * GS, GS)],
                                 dst_v.at[nib], sem_i)

            g_start(ib, 0, 0)
            for j in range(GS - 1):
                g_wait(ib, j, j % 2)
                g_start(ib, j + 1, (j + 1) % 2)
                scat(ib, j, j % 2)
            g_wait(ib, GS - 1, (GS - 1) % 2)
            scat(ib, GS - 1, (GS - 1) % 2)

            @pl.when(g + 1 < ng)
            def _prefetch_wait():
                pltpu.make_async_copy(
                    src_hbm.at[wid, pl.ds((g + 1) * GS, GS)],
                    src_v.at[nib], sem_i).wait()
                pltpu.make_async_copy(
                    dst_hbm.at[wid, pl.ds((g + 1) * GS, GS)],
                    dst_v.at[nib], sem_i).wait()

            return 0

        lax.fori_loop(0, ng, group, 0)
        plsc.subcore_barrier()

        pltpu.sync_copy(acc.at[pl.ds(sid * SP, SP)],
                        out_hbm.at[cid, pl.ds(sid * SP, SP)])

    return seg_sum(table, src_w, dst_w, init)


def _combine_body(p_ref, w_ref, b_ref, a_ref, o_ref):
    x = p_ref[...]
    s = x[0] + x[1]
    agg = s[:, :D]
    deg = s[:, D:D + 1]
    y = jnp.dot(agg / deg, w_ref[...], preferred_element_type=jnp.float32)
    y = y + b_ref[...]
    a = a_ref[0, 0]
    o_ref[...] = jnp.where(y > 0, y, a * y)


def _tc_combine(parts, W, b, prelu_a):
    R = 1000
    grid = (N // R,)
    return pl.pallas_call(
        _combine_body,
        grid=grid,
        in_specs=[
            pl.BlockSpec((NC, R, DA), lambda i: (0, i, 0)),
            pl.BlockSpec((D, D), lambda i: (0, 0)),
            pl.BlockSpec((1, D), lambda i: (0, 0)),
            pl.BlockSpec((1, 1), lambda i: (0, 0)),
        ],
        out_specs=pl.BlockSpec((R, D), lambda i: (i, 0)),
        out_shape=jax.ShapeDtypeStruct((N, D), jnp.float32),
    )(parts, W, b.reshape(1, D), prelu_a.reshape(1, 1))


def kernel(feat, edge_index, feat_dst, W, b, prelu_a):
    E = edge_index.shape[1]
    ew = -(-E // NW)
    kc = -(-ew // (C * GS)) * GS
    ep = NW * kc * C

    src = edge_index[0]
    dst = edge_index[1]
    src_p = jnp.concatenate(
        [src, jnp.full((ep - E,), N, jnp.int32)]).reshape(NW, kc, C)
    dst_p = jnp.concatenate(
        [dst, jnp.arange(ep - E, dtype=jnp.int32) % NP]).reshape(NW, kc, C)

    ones_col = jnp.ones((N, 1), jnp.float32)
    zpad = jnp.zeros((N, DA - D - 1), jnp.float32)
    table = jnp.concatenate([feat, ones_col, zpad], axis=1)
    table = jnp.pad(table, ((0, 8), (0, 0)))
    init0 = jnp.concatenate([feat_dst, ones_col, zpad], axis=1)
    init0 = jnp.pad(init0, ((0, NP - N), (0, 0)))
    init = jnp.stack([init0, jnp.zeros_like(init0)])

    parts = _sc_segment_sum(table, src_p, dst_p, init)
    return _tc_combine(parts, W, b, jnp.asarray(prelu_a, jnp.float32))


# speedup vs baseline: 1.2358x; 1.2358x over previous
import functools

import jax
import jax.numpy as jnp
from jax import lax
from jax.experimental import pallas as pl
from jax.experimental.pallas import tpu as pltpu
from jax.experimental.pallas import tpu_sc as plsc

N = 10000
D = 128
DA = 136
NC = 2
NS = 16
NW = NC * NS
C = 128
GS = 8
NP = 10112
SP = NP // NS


def _sc_segment_sum(table, src_w, dst_w, init):
    kc = src_w.shape[1]
    ng = kc // GS
    mesh = plsc.VectorSubcoreMesh(core_axis_name="c", subcore_axis_name="s")

    @functools.partial(
        pl.kernel,
        out_type=jax.ShapeDtypeStruct((NC, NP, DA), jnp.float32),
        mesh=mesh,
        compiler_params=pltpu.CompilerParams(use_tc_tiling_on_sc=False),
        scratch_types=[
            pltpu.VMEM((2, GS, C), jnp.int32),
            pltpu.VMEM((2, GS, C), jnp.int32),
            pltpu.VMEM((2, C, DA), jnp.float32),
            pltpu.VMEM_SHARED((NP, DA), jnp.float32),
            pltpu.SemaphoreType.DMA,
            pltpu.SemaphoreType.DMA,
        ],
    )
    def seg_sum(table_hbm, src_hbm, dst_hbm, init_hbm, out_hbm,
                src_v, dst_v, rows_v, acc, sem, sem_i):
        cid = lax.axis_index("c")
        sid = lax.axis_index("s")
        wid = cid * NS + sid

        pltpu.sync_copy(init_hbm.at[cid, pl.ds(sid * SP, SP)],
                        acc.at[pl.ds(sid * SP, SP)])
        pltpu.sync_copy(src_hbm.at[wid, pl.ds(0, GS)], src_v.at[0])
        pltpu.sync_copy(dst_hbm.at[wid, pl.ds(0, GS)], dst_v.at[0])
        plsc.subcore_barrier()

        def g_start(ib, j, buf):
            pltpu.async_copy(table_hbm.at[src_v.at[ib, j]], rows_v.at[buf],
                             sem)

        def g_wait(ib, j, buf):
            pltpu.make_async_copy(
                table_hbm.at[src_v.at[ib, j]], rows_v.at[buf], sem).wait()

        def scat(ib, j, buf):
            pltpu.sync_copy(rows_v.at[buf], acc.at[dst_v.at[ib, j]], add=True)

        def group(g, _):
            ib = lax.rem(g, 2)
            nib = 1 - ib

            @pl.when(g + 1 < ng)
            def _prefetch():
                pltpu.async_copy(src_hbm.at[wid, pl.ds((g + 1) * GS, GS)],
                                 src_v.at[nib], sem_i)
                pltpu.async_copy(dst_hbm.at[wid, pl.ds((g + 1) * GS, GS)],
                                 dst_v.at[nib], sem_i)

            g_start(ib, 0, 0)
            for j in range(GS - 1):
                g_wait(ib, j, j % 2)
                g_start(ib, j + 1, (j + 1) % 2)
                scat(ib, j, j % 2)
            g_wait(ib, GS - 1, (GS - 1) % 2)
            scat(ib, GS - 1, (GS - 1) % 2)

            @pl.when(g + 1 < ng)
            def _prefetch_wait():
                pltpu.make_async_copy(
                    src_hbm.at[wid, pl.ds((g + 1) * GS, GS)],
                    src_v.at[nib], sem_i).wait()
                pltpu.make_async_copy(
                    dst_hbm.at[wid, pl.ds((g + 1) * GS, GS)],
                    dst_v.at[nib], sem_i).wait()

            return 0

        lax.fori_loop(0, ng, group, 0)
        plsc.subcore_barrier()

        pltpu.sync_copy(acc.at[pl.ds(sid * SP, SP)],
                        out_hbm.at[cid, pl.ds(sid * SP, SP)])

    return seg_sum(table, src_w, dst_w, init)


def _combine_body(p_ref, w_ref, b_ref, a_ref, o_ref):
    x = p_ref[...]
    s = x[0] + x[1]
    agg = s[:, :D]
    deg = s[:, D:D + 1]
    y = jnp.dot(agg / deg, w_ref[...], preferred_element_type=jnp.float32)
    y = y + b_ref[...]
    a = a_ref[0, 0]
    o_ref[...] = jnp.where(y > 0, y, a * y)


def _tc_combine(parts, W, b, prelu_a):
    R = 1000
    grid = (N // R,)
    return pl.pallas_call(
        _combine_body,
        grid=grid,
        in_specs=[
            pl.BlockSpec((NC, R, DA), lambda i: (0, i, 0)),
            pl.BlockSpec((D, D), lambda i: (0, 0)),
            pl.BlockSpec((1, D), lambda i: (0, 0)),
            pl.BlockSpec((1, 1), lambda i: (0, 0)),
        ],
        out_specs=pl.BlockSpec((R, D), lambda i: (i, 0)),
        out_shape=jax.ShapeDtypeStruct((N, D), jnp.float32),
    )(parts, W, b.reshape(1, D), prelu_a.reshape(1, 1))


def kernel(feat, edge_index, feat_dst, W, b, prelu_a):
    E = edge_index.shape[1]
    ew = -(-E // NW)
    kc = -(-ew // (C * GS)) * GS
    ep = NW * kc * C

    src = edge_index[0]
    dst = edge_index[1]
    src_p = jnp.concatenate(
        [src, jnp.full((ep - E,), N, jnp.int32)]
    ).reshape(kc * C, NW).T.reshape(NW, kc, C)
    dst_p = jnp.concatenate(
        [dst, jnp.arange(ep - E, dtype=jnp.int32) % NP]
    ).reshape(kc * C, NW).T.reshape(NW, kc, C)

    ones_col = jnp.ones((N, 1), jnp.float32)
    zpad = jnp.zeros((N, DA - D - 1), jnp.float32)
    table = jnp.concatenate([feat, ones_col, zpad], axis=1)
    table = jnp.pad(table, ((0, 8), (0, 0)))
    init0 = jnp.concatenate([feat_dst, ones_col, zpad], axis=1)
    init0 = jnp.pad(init0, ((0, NP - N), (0, 0)))
    init = jnp.stack([init0, jnp.zeros_like(init0)])

    parts = _sc_segment_sum(table, src_p, dst_p, init)
    return _tc_combine(parts, W, b, jnp.asarray(prelu_a, jnp.float32))


# speedup vs baseline: 1.3310x; 1.0771x over previous
import functools

import jax
import jax.numpy as jnp
from jax import lax
from jax.experimental import pallas as pl
from jax.experimental.pallas import tpu as pltpu
from jax.experimental.pallas import tpu_sc as plsc

N = 10000
D = 128
DA = 136
NC = 2
NS = 16
NW = NC * NS
C = 128
GS = 8
NP = 10112
SP = NP // NS


def _sc_segment_sum(table, src_w, dst_w):
    kc = src_w.shape[1]
    ng = kc // GS
    mesh = plsc.VectorSubcoreMesh(core_axis_name="c", subcore_axis_name="s")

    @functools.partial(
        pl.kernel,
        out_type=jax.ShapeDtypeStruct((NC, NP, DA), jnp.float32),
        mesh=mesh,
        compiler_params=pltpu.CompilerParams(use_tc_tiling_on_sc=False),
        scratch_types=[
            pltpu.VMEM((2, GS, C), jnp.int32),
            pltpu.VMEM((2, GS, C), jnp.int32),
            pltpu.VMEM((2, C, DA), jnp.float32),
            pltpu.VMEM_SHARED((NP, DA), jnp.float32),
            pltpu.SemaphoreType.DMA,
            pltpu.SemaphoreType.DMA,
        ],
    )
    def seg_sum(table_hbm, src_hbm, dst_hbm, out_hbm,
                src_v, dst_v, rows_v, acc, sem, sem_i):
        cid = lax.axis_index("c")
        sid = lax.axis_index("s")
        wid = cid * NS + sid

        zeros16 = jnp.zeros((16,), jnp.float32)

        def zrow(i, _):
            for j in range(8):
                rows_v[0, i, pl.ds(j * 16, 16)] = zeros16
            rows_v[0, i, pl.ds(DA - 16, 16)] = zeros16
            return 0

        lax.fori_loop(0, C, zrow, 0)
        base = sid * SP
        for r in range(SP // C):
            pltpu.sync_copy(rows_v.at[0], acc.at[pl.ds(base + r * C, C)])
        rem = SP % C
        if rem:
            pltpu.sync_copy(rows_v.at[0, pl.ds(0, rem)],
                            acc.at[pl.ds(base + (SP // C) * C, rem)])
        pltpu.sync_copy(src_hbm.at[wid, pl.ds(0, GS)], src_v.at[0])
        pltpu.sync_copy(dst_hbm.at[wid, pl.ds(0, GS)], dst_v.at[0])
        plsc.subcore_barrier()

        def g_start(ib, j, buf):
            pltpu.async_copy(table_hbm.at[src_v.at[ib, j]], rows_v.at[buf],
                             sem)

        def g_wait(ib, j, buf):
            pltpu.make_async_copy(
                table_hbm.at[src_v.at[ib, j]], rows_v.at[buf], sem).wait()

        def scat(ib, j, buf):
            pltpu.sync_copy(rows_v.at[buf], acc.at[dst_v.at[ib, j]], add=True)

        def group(g, _):
            ib = lax.rem(g, 2)
            nib = 1 - ib

            @pl.when(g + 1 < ng)
            def _prefetch():
                pltpu.async_copy(src_hbm.at[wid, pl.ds((g + 1) * GS, GS)],
                                 src_v.at[nib], sem_i)
                pltpu.async_copy(dst_hbm.at[wid, pl.ds((g + 1) * GS, GS)],
                                 dst_v.at[nib], sem_i)

            g_start(ib, 0, 0)
            for j in range(GS - 1):
                g_wait(ib, j, j % 2)
                g_start(ib, j + 1, (j + 1) % 2)
                scat(ib, j, j % 2)
            g_wait(ib, GS - 1, (GS - 1) % 2)
            scat(ib, GS - 1, (GS - 1) % 2)

            @pl.when(g + 1 < ng)
            def _prefetch_wait():
                pltpu.make_async_copy(
                    src_hbm.at[wid, pl.ds((g + 1) * GS, GS)],
                    src_v.at[nib], sem_i).wait()
                pltpu.make_async_copy(
                    dst_hbm.at[wid, pl.ds((g + 1) * GS, GS)],
                    dst_v.at[nib], sem_i).wait()

            return 0

        lax.fori_loop(0, ng, group, 0)
        plsc.subcore_barrier()

        pltpu.sync_copy(acc.at[pl.ds(sid * SP, SP)],
                        out_hbm.at[cid, pl.ds(sid * SP, SP)])

    return seg_sum(table, src_w, dst_w)


def _combine_body(p_ref, fd_ref, w_ref, b_ref, a_ref, o_ref):
    x = p_ref[...]
    s = x[0] + x[1]
    agg = s[:, :D] + fd_ref[...]
    deg = s[:, D:D + 1] + 1.0
    y = jnp.dot(agg / deg, w_ref[...], preferred_element_type=jnp.float32)
    y = y + b_ref[...]
    a = a_ref[0, 0]
    o_ref[...] = jnp.where(y > 0, y, a * y)


def _tc_combine(parts, feat_dst, W, b, prelu_a):
    R = 1000
    grid = (N // R,)
    return pl.pallas_call(
        _combine_body,
        grid=grid,
        in_specs=[
            pl.BlockSpec((NC, R, DA), lambda i: (0, i, 0)),
            pl.BlockSpec((R, D), lambda i: (i, 0)),
            pl.BlockSpec((D, D), lambda i: (0, 0)),
            pl.BlockSpec((1, D), lambda i: (0, 0)),
            pl.BlockSpec((1, 1), lambda i: (0, 0)),
        ],
        out_specs=pl.BlockSpec((R, D), lambda i: (i, 0)),
        out_shape=jax.ShapeDtypeStruct((N, D), jnp.float32),
    )(parts, feat_dst, W, b.reshape(1, D), prelu_a.reshape(1, 1))


def kernel(feat, edge_index, feat_dst, W, b, prelu_a):
    E = edge_index.shape[1]
    ew = -(-E // NW)
    kc = -(-ew // (C * GS)) * GS
    ep = NW * kc * C

    src = edge_index[0]
    dst = edge_index[1]
    src_p = jnp.concatenate(
        [src, jnp.full((ep - E,), N, jnp.int32)]
    ).reshape(kc * C, NW).T.reshape(NW, kc, C)
    dst_p = jnp.concatenate(
        [dst, jnp.arange(ep - E, dtype=jnp.int32) % NP]
    ).reshape(kc * C, NW).T.reshape(NW, kc, C)

    ones_col = jnp.ones((N, 1), jnp.float32)
    zpad = jnp.zeros((N, DA - D - 1), jnp.float32)
    table = jnp.concatenate([feat, ones_col, zpad], axis=1)
    table = jnp.pad(table, ((0, 8), (0, 0)))

    parts = _sc_segment_sum(table, src_p, dst_p)
    return _tc_combine(parts, feat_dst, W, b,
                       jnp.asarray(prelu_a, jnp.float32))


# speedup vs baseline: 1.6712x; 1.2556x over previous
import functools

import jax
import jax.numpy as jnp
from jax import lax
from jax.experimental import pallas as pl
from jax.experimental.pallas import tpu as pltpu
from jax.experimental.pallas import tpu_sc as plsc

N = 10000
D = 128
DA = 136
NC = 2
NS = 16
NW = NC * NS
C = 128
GS = 8
NP = 10112
SP = NP // NS


def _sc_segment_sum(table, src_w, dst_w):
    kc = src_w.shape[1]
    mesh = plsc.VectorSubcoreMesh(core_axis_name="c", subcore_axis_name="s")

    @functools.partial(
        pl.kernel,
        out_type=jax.ShapeDtypeStruct((NC, NP, DA), jnp.float32),
        mesh=mesh,
        compiler_params=pltpu.CompilerParams(use_tc_tiling_on_sc=False),
        scratch_types=[
            pltpu.VMEM((kc, C), jnp.int32),
            pltpu.VMEM((kc, C), jnp.int32),
            pltpu.VMEM((C, DA), jnp.float32),
            pltpu.VMEM_SHARED((NP, DA), jnp.float32),
            pltpu.SemaphoreType.DMA,
        ],
    )
    def seg_sum(table_hbm, src_hbm, dst_hbm, out_hbm,
                src_v, dst_v, rows_v, acc, sem):
        cid = lax.axis_index("c")
        sid = lax.axis_index("s")
        wid = cid * NS + sid

        zeros16 = jnp.zeros((16,), jnp.float32)

        def zrow(i, _):
            for j in range(8):
                rows_v[i, pl.ds(j * 16, 16)] = zeros16
            rows_v[i, pl.ds(DA - 16, 16)] = zeros16
            return 0

        lax.fori_loop(0, C, zrow, 0)
        base = sid * SP
        for r in range(SP // C):
            pltpu.sync_copy(rows_v, acc.at[pl.ds(base + r * C, C)])
        rem = SP % C
        if rem:
            pltpu.sync_copy(rows_v.at[pl.ds(0, rem)],
                            acc.at[pl.ds(base + (SP // C) * C, rem)])
        pltpu.sync_copy(src_hbm.at[wid], src_v)
        pltpu.sync_copy(dst_hbm.at[wid], dst_v)
        plsc.subcore_barrier()

        def chunk(k, _):
            pltpu.async_copy(table_hbm.at[src_v.at[k]], rows_v, sem).wait()
            pltpu.sync_copy(rows_v, acc.at[dst_v.at[k]], add=True)
            return 0

        lax.fori_loop(0, kc, chunk, 0)
        plsc.subcore_barrier()

        pltpu.sync_copy(acc.at[pl.ds(sid * SP, SP)],
                        out_hbm.at[cid, pl.ds(sid * SP, SP)])

    return seg_sum(table, src_w, dst_w)


def _combine_body(p_ref, fd_ref, w_ref, b_ref, a_ref, o_ref):
    x = p_ref[...]
    s = x[0] + x[1]
    agg = s[:, :D] + fd_ref[...]
    deg = s[:, D:D + 1] + 1.0
    y = jnp.dot(agg / deg, w_ref[...], preferred_element_type=jnp.float32)
    y = y + b_ref[...]
    a = a_ref[0, 0]
    o_ref[...] = jnp.where(y > 0, y, a * y)


def _tc_combine(parts, feat_dst, W, b, prelu_a):
    R = 1000
    grid = (N // R,)
    return pl.pallas_call(
        _combine_body,
        grid=grid,
        in_specs=[
            pl.BlockSpec((NC, R, DA), lambda i: (0, i, 0)),
            pl.BlockSpec((R, D), lambda i: (i, 0)),
            pl.BlockSpec((D, D), lambda i: (0, 0)),
            pl.BlockSpec((1, D), lambda i: (0, 0)),
            pl.BlockSpec((1, 1), lambda i: (0, 0)),
        ],
        out_specs=pl.BlockSpec((R, D), lambda i: (i, 0)),
        out_shape=jax.ShapeDtypeStruct((N, D), jnp.float32),
    )(parts, feat_dst, W, b.reshape(1, D), prelu_a.reshape(1, 1))


def kernel(feat, edge_index, feat_dst, W, b, prelu_a):
    E = edge_index.shape[1]
    ew = -(-E // NW)
    kc = -(-ew // C)
    ep = NW * kc * C

    src = edge_index[0]
    dst = edge_index[1]
    src_p = jnp.concatenate(
        [src, jnp.full((ep - E,), N, jnp.int32)]
    ).reshape(kc * C, NW).T.reshape(NW, kc, C)
    dst_p = jnp.concatenate(
        [dst, jnp.arange(ep - E, dtype=jnp.int32) % NP]
    ).reshape(kc * C, NW).T.reshape(NW, kc, C)

    ones_col = jnp.ones((N, 1), jnp.float32)
    zpad = jnp.zeros((N, DA - D - 1), jnp.float32)
    table = jnp.concatenate([feat, ones_col, zpad], axis=1)
    table = jnp.pad(table, ((0, 8), (0, 0)))

    parts = _sc_segment_sum(table, src_p, dst_p)
    return _tc_combine(parts, feat_dst, W, b,
                       jnp.asarray(prelu_a, jnp.float32))
